# baseline (speedup 1.0000x reference)
import jax
import jax.numpy as jnp
from jax.experimental import pallas as pl

_STRIDES = (8.0, 16.0, 32.0)
_WLOG = (6, 5, 4)
_NS = (4096, 1024, 256)
_NTOT = 5376
_C = 85
_CH = 256


def _decode_chunk(v, stride, wlog, row0):
    n = v.shape[0]
    p = row0 + jax.lax.broadcasted_iota(jnp.int32, (n, 1), 0)
    gx = (p & ((1 << wlog) - 1)).astype(jnp.float32)
    gy = (p >> wlog).astype(jnp.float32)
    c = jax.lax.broadcasted_iota(jnp.int32, (n, _C), 1)
    g = jnp.where(c == 0, gx, gy)
    xy = (v + g) * stride
    wh = jnp.exp(v) * stride
    return jnp.where(c < 2, xy, jnp.where(c < 4, wh, v))


def _decode_kernel(f0_ref, f1_ref, f2_ref, out_ref):
    j = pl.program_id(1)

    @pl.when(j < 16)
    def _():
        out_ref[0] = _decode_chunk(f0_ref[0], 8.0, 6, j * _CH)

    @pl.when(jnp.logical_and(j >= 16, j < 20))
    def _():
        out_ref[0] = _decode_chunk(f1_ref[0], 16.0, 5, (j - 16) * _CH)

    @pl.when(j >= 20)
    def _():
        out_ref[0] = _decode_chunk(f2_ref[0], 32.0, 4, 0)


@jax.jit
def kernel(feat0, feat1, feat2, targets):
    b = feat0.shape[0]
    f0 = feat0.reshape(b, _NS[0], _C)
    f1 = feat1.reshape(b, _NS[1], _C)
    f2 = feat2.reshape(b, _NS[2], _C)
    return pl.pallas_call(
        _decode_kernel,
        grid=(b, 21),
        in_specs=[
            pl.BlockSpec((1, _CH, _C), lambda i, j: (i, jnp.minimum(j, 15), 0)),
            pl.BlockSpec(
                (1, _CH, _C),
                lambda i, j: (i, jnp.clip(j - 16, 0, 3), 0),
            ),
            pl.BlockSpec((1, _CH, _C), lambda i, j: (i, 0, 0)),
        ],
        out_specs=pl.BlockSpec((1, _CH, _C), lambda i, j: (i, j, 0)),
        out_shape=jax.ShapeDtypeStruct((b, _NTOT, _C), jnp.float32),
    )(f0, f1, f2)


# speedup vs baseline: 3.3442x; 3.3442x over previous
import jax
import jax.numpy as jnp
from jax.experimental import pallas as pl

_STRIDES = (8.0, 16.0, 32.0)
_WS = (64, 32, 16)
_NS = (4096, 1024, 256)
_OFFS = (0, 4096, 5120)
_NTOT = 5376
_C = 85
_BB = 2


def _decode_level(v, stride, w):
    n = v.shape[1]
    p = jax.lax.broadcasted_iota(jnp.int32, (1, n, 1), 1)
    gx = (p & (w - 1)).astype(jnp.float32)
    gy = (p // w).astype(jnp.float32)
    c = jax.lax.broadcasted_iota(jnp.int32, (1, n, _C), 2)
    g = jnp.where(c == 0, gx, gy)
    xy = (v + g) * stride
    wh = jnp.exp(v) * stride
    return jnp.where(c < 2, xy, jnp.where(c < 4, wh, v))


def _decode_kernel(f0_ref, f1_ref, f2_ref, out_ref):
    for ref, stride, w, off, n in zip(
        (f0_ref, f1_ref, f2_ref), _STRIDES, _WS, _OFFS, _NS
    ):
        out_ref[:, pl.ds(off, n), :] = _decode_level(ref[:], stride, w)


@jax.jit
def kernel(feat0, feat1, feat2, targets):
    b = feat0.shape[0]
    f0 = feat0.reshape(b, _NS[0], _C)
    f1 = feat1.reshape(b, _NS[1], _C)
    f2 = feat2.reshape(b, _NS[2], _C)
    return pl.pallas_call(
        _decode_kernel,
        grid=(b // _BB,),
        in_specs=[
            pl.BlockSpec((_BB, _NS[0], _C), lambda i: (i, 0, 0)),
            pl.BlockSpec((_BB, _NS[1], _C), lambda i: (i, 0, 0)),
            pl.BlockSpec((_BB, _NS[2], _C), lambda i: (i, 0, 0)),
        ],
        out_specs=pl.BlockSpec((_BB, _NTOT, _C), lambda i: (i, 0, 0)),
        out_shape=jax.ShapeDtypeStruct((b, _NTOT, _C), jnp.float32),
    )(f0, f1, f2)


# speedup vs baseline: 3.3602x; 1.0048x over previous
import jax
import jax.numpy as jnp
from jax.experimental import pallas as pl

_STRIDES = (8.0, 16.0, 32.0)
_WS = (64, 32, 16)
_NS = (4096, 1024, 256)
_OFFS = (0, 4096, 5120)
_NTOT = 5376
_C = 85
_BB = 4


def _decode_level(v, stride, w):
    n = v.shape[1]
    p = jax.lax.broadcasted_iota(jnp.int32, (1, n, 1), 1)
    gx = (p & (w - 1)).astype(jnp.float32)
    gy = (p // w).astype(jnp.float32)
    c = jax.lax.broadcasted_iota(jnp.int32, (1, n, _C), 2)
    g = jnp.where(c == 0, gx, gy)
    xy = (v + g) * stride
    wh = jnp.exp(v) * stride
    return jnp.where(c < 2, xy, jnp.where(c < 4, wh, v))


def _decode_kernel(f0_ref, f1_ref, f2_ref, out_ref):
    for ref, stride, w, off, n in zip(
        (f0_ref, f1_ref, f2_ref), _STRIDES, _WS, _OFFS, _NS
    ):
        out_ref[:, pl.ds(off, n), :] = _decode_level(ref[:], stride, w)


@jax.jit
def kernel(feat0, feat1, feat2, targets):
    b = feat0.shape[0]
    f0 = feat0.reshape(b, _NS[0], _C)
    f1 = feat1.reshape(b, _NS[1], _C)
    f2 = feat2.reshape(b, _NS[2], _C)
    return pl.pallas_call(
        _decode_kernel,
        grid=(b // _BB,),
        in_specs=[
            pl.BlockSpec((_BB, _NS[0], _C), lambda i: (i, 0, 0)),
            pl.BlockSpec((_BB, _NS[1], _C), lambda i: (i, 0, 0)),
            pl.BlockSpec((_BB, _NS[2], _C), lambda i: (i, 0, 0)),
        ],
        out_specs=pl.BlockSpec((_BB, _NTOT, _C), lambda i: (i, 0, 0)),
        out_shape=jax.ShapeDtypeStruct((b, _NTOT, _C), jnp.float32),
    )(f0, f1, f2)
